# baseline (speedup 1.0000x reference)
import functools
import math

import jax
import jax.numpy as jnp
from jax import lax
from jax.experimental import pallas as pl
from jax.experimental.pallas import tpu as pltpu
from jax.experimental.pallas import tpu_sc as plsc

D_MODEL = 768
SEQ_LEN = 4096
BATCH = 4
N_FLAT = SEQ_LEN * BATCH

NUM_WORKERS = 32
PER_WORKER = N_FLAT // NUM_WORKERS
CHUNK = 64
NCHUNK = PER_WORKER // CHUNK
POS_PER_CHUNK = CHUNK // BATCH
LANES = 16
VREGS_PER_ROW = D_MODEL // LANES
SCALE = math.sqrt(D_MODEL)


def _make_kernel():
    mesh = plsc.VectorSubcoreMesh(core_axis_name="c", subcore_axis_name="s")

    @functools.partial(
        pl.kernel,
        mesh=mesh,
        out_type=jax.ShapeDtypeStruct((NUM_WORKERS, NCHUNK, CHUNK, D_MODEL),
                                      jnp.float32),
        scratch_types=[
            pltpu.VMEM((NCHUNK, CHUNK), jnp.int32),
            pltpu.VMEM((POS_PER_CHUNK, D_MODEL), jnp.float32),
            pltpu.VMEM((CHUNK, D_MODEL), jnp.float32),
            pltpu.SemaphoreType.DMA,
        ],
    )
    def k(x_hbm, table_hbm, pe_hbm, out_hbm, idx_v, pe_v, buf_v, sem):
        cid = lax.axis_index("c")
        sid = lax.axis_index("s")
        wid = sid * 2 + cid

        pltpu.sync_copy(x_hbm.at[wid], idx_v)

        def chunk_body(c, _):
            g = pltpu.async_copy(table_hbm.at[idx_v.at[c]], buf_v, sem)
            pltpu.sync_copy(pe_hbm.at[wid, c], pe_v)
            g.wait()

            def pos_body(p, _):
                def vreg_body(j, _):
                    off = j * LANES
                    pvec = pe_v[p, pl.ds(off, LANES)]
                    for b in range(BATCH):
                        r = p * BATCH + b
                        buf_v[r, pl.ds(off, LANES)] = (
                            buf_v[r, pl.ds(off, LANES)] * SCALE + pvec)
                    return 0

                return lax.fori_loop(0, VREGS_PER_ROW, vreg_body, 0)

            lax.fori_loop(0, POS_PER_CHUNK, pos_body, 0)
            pltpu.sync_copy(buf_v, out_hbm.at[wid, c])
            return 0

        lax.fori_loop(0, NCHUNK, chunk_body, 0)

    return k


_sc_kernel = _make_kernel()


def kernel(x, table, pe):
    xf = jnp.asarray(x, jnp.int32).reshape(NUM_WORKERS, NCHUNK, CHUNK)
    pef = pe.reshape(-1, D_MODEL)[:SEQ_LEN].reshape(
        NUM_WORKERS, NCHUNK, POS_PER_CHUNK, D_MODEL)
    out = _sc_kernel(xf, table, pef)
    return out.reshape(SEQ_LEN, BATCH, D_MODEL)


# speedup vs baseline: 1.2671x; 1.2671x over previous
import functools
import math

import jax
import jax.numpy as jnp
from jax import lax
from jax.experimental import pallas as pl
from jax.experimental.pallas import tpu as pltpu
from jax.experimental.pallas import tpu_sc as plsc

D_MODEL = 768
SEQ_LEN = 4096
BATCH = 4
N_FLAT = SEQ_LEN * BATCH

NUM_WORKERS = 32
PER_WORKER = N_FLAT // NUM_WORKERS
CHUNK = 32
NCHUNK = PER_WORKER // CHUNK
POS_PER_CHUNK = CHUNK // BATCH
NSLOT = 4
LANES = 16
VREGS_PER_ROW = D_MODEL // LANES
SCALE = math.sqrt(D_MODEL)


def _make_kernel():
    mesh = plsc.VectorSubcoreMesh(core_axis_name="c", subcore_axis_name="s")

    @functools.partial(
        pl.kernel,
        mesh=mesh,
        out_type=jax.ShapeDtypeStruct((NUM_WORKERS, NCHUNK, CHUNK, D_MODEL),
                                      jnp.float32),
        scratch_types=[
            pltpu.VMEM((NCHUNK, CHUNK), jnp.int32),
            pltpu.VMEM((NSLOT, POS_PER_CHUNK, D_MODEL), jnp.float32),
            pltpu.VMEM((NSLOT, CHUNK, D_MODEL), jnp.float32),
            pltpu.SemaphoreType.DMA((NSLOT,)),
            pltpu.SemaphoreType.DMA((NSLOT,)),
        ],
    )
    def k(x_hbm, table_hbm, pe_hbm, out_hbm, idx_v, pe_v, buf_v, isem, osem):
        cid = lax.axis_index("c")
        sid = lax.axis_index("s")
        wid = sid * 2 + cid

        pltpu.sync_copy(x_hbm.at[wid], idx_v)

        def fire_in(c):
            s = c % NSLOT
            g = pltpu.async_copy(table_hbm.at[idx_v.at[c]], buf_v.at[s],
                                 isem.at[s])
            p = pltpu.async_copy(pe_hbm.at[wid, c], pe_v.at[s], isem.at[s])
            return g, p

        def compute(s):
            def j_body(j, _):
                off = j * LANES
                for p in range(POS_PER_CHUNK):
                    pvec = pe_v[s, p, pl.ds(off, LANES)]
                    for b in range(BATCH):
                        r = p * BATCH + b
                        buf_v[s, r, pl.ds(off, LANES)] = (
                            buf_v[s, r, pl.ds(off, LANES)] * SCALE + pvec)
                return 0

            lax.fori_loop(0, VREGS_PER_ROW, j_body, 0)

        in_flight = {}
        store_flight = {}
        for c in range(2):
            in_flight[c] = fire_in(c)

        for c in range(NCHUNK):
            s = c % NSLOT
            if c + 2 < NCHUNK:
                if c - 2 >= 0:
                    store_flight.pop(c - 2).wait()
                in_flight[c + 2] = fire_in(c + 2)
            g, p = in_flight.pop(c)
            g.wait()
            p.wait()
            compute(s)
            store_flight[c] = pltpu.async_copy(buf_v.at[s],
                                               out_hbm.at[wid, c], osem.at[s])

        store_flight.pop(NCHUNK - 2).wait()
        store_flight.pop(NCHUNK - 1).wait()

    return k


_sc_kernel = _make_kernel()


def kernel(x, table, pe):
    xf = jnp.asarray(x, jnp.int32).reshape(NUM_WORKERS, NCHUNK, CHUNK)
    pef = pe.reshape(-1, D_MODEL)[:SEQ_LEN].reshape(
        NUM_WORKERS, NCHUNK, POS_PER_CHUNK, D_MODEL)
    out = _sc_kernel(xf, table, pef)
    return out.reshape(SEQ_LEN, BATCH, D_MODEL)


# speedup vs baseline: 1.2710x; 1.0031x over previous
import functools
import math

import jax
import jax.numpy as jnp
from jax import lax
from jax.experimental import pallas as pl
from jax.experimental.pallas import tpu as pltpu
from jax.experimental.pallas import tpu_sc as plsc

D_MODEL = 768
SEQ_LEN = 4096
BATCH = 4
N_FLAT = SEQ_LEN * BATCH

NUM_WORKERS = 32
PER_WORKER = N_FLAT // NUM_WORKERS
CHUNK = 32
NCHUNK = PER_WORKER // CHUNK
POS_PER_CHUNK = CHUNK // BATCH
NSLOT = 4
LANES = 16
VREGS_PER_ROW = D_MODEL // LANES
SCALE = math.sqrt(D_MODEL)


def _make_kernel():
    mesh = plsc.VectorSubcoreMesh(core_axis_name="c", subcore_axis_name="s")

    @functools.partial(
        pl.kernel,
        mesh=mesh,
        out_type=jax.ShapeDtypeStruct((N_FLAT, D_MODEL), jnp.float32),
        scratch_types=[
            pltpu.VMEM((NCHUNK, CHUNK), jnp.int32),
            pltpu.VMEM((NSLOT, POS_PER_CHUNK, D_MODEL), jnp.float32),
            pltpu.VMEM((NSLOT, CHUNK, D_MODEL), jnp.float32),
            pltpu.SemaphoreType.DMA((NSLOT,)),
            pltpu.SemaphoreType.DMA((NSLOT,)),
        ],
    )
    def k(x_hbm, table_hbm, pe_hbm, out_hbm, idx_v, pe_v, buf_v, isem, osem):
        cid = lax.axis_index("c")
        sid = lax.axis_index("s")
        wid = sid * 2 + cid

        pltpu.sync_copy(x_hbm.at[wid], idx_v)

        def fire_in(c):
            s = c % NSLOT
            g = pltpu.async_copy(table_hbm.at[idx_v.at[c]], buf_v.at[s],
                                 isem.at[s])
            p = pltpu.async_copy(
                pe_hbm.at[pl.ds(wid * (PER_WORKER // BATCH)
                                + c * POS_PER_CHUNK, POS_PER_CHUNK)],
                pe_v.at[s], isem.at[s])
            return g, p

        def compute(s):
            def j_body(j, _):
                off = j * LANES
                for p in range(POS_PER_CHUNK):
                    pvec = pe_v[s, p, pl.ds(off, LANES)]
                    for b in range(BATCH):
                        r = p * BATCH + b
                        buf_v[s, r, pl.ds(off, LANES)] = (
                            buf_v[s, r, pl.ds(off, LANES)] * SCALE + pvec)
                return 0

            lax.fori_loop(0, VREGS_PER_ROW, j_body, 0)

        in_flight = {}
        store_flight = {}
        for c in range(2):
            in_flight[c] = fire_in(c)

        for c in range(NCHUNK):
            s = c % NSLOT
            if c + 2 < NCHUNK:
                if c - 2 >= 0:
                    store_flight.pop(c - 2).wait()
                in_flight[c + 2] = fire_in(c + 2)
            g, p = in_flight.pop(c)
            g.wait()
            p.wait()
            compute(s)
            store_flight[c] = pltpu.async_copy(
                buf_v.at[s],
                out_hbm.at[pl.ds(wid * PER_WORKER + c * CHUNK, CHUNK)],
                osem.at[s])

        store_flight.pop(NCHUNK - 2).wait()
        store_flight.pop(NCHUNK - 1).wait()

    return k


_sc_kernel = _make_kernel()


def kernel(x, table, pe):
    xf = jnp.asarray(x, jnp.int32).reshape(NUM_WORKERS, NCHUNK, CHUNK)
    pef = pe.reshape(-1, D_MODEL)
    out = _sc_kernel(xf, table, pef)
    return out.reshape(SEQ_LEN, BATCH, D_MODEL)


# speedup vs baseline: 2.2301x; 1.7546x over previous
import functools
import math

import jax
import jax.numpy as jnp
from jax import lax
from jax.experimental import pallas as pl
from jax.experimental.pallas import tpu as pltpu
from jax.experimental.pallas import tpu_sc as plsc

D_MODEL = 768
SEQ_LEN = 4096
BATCH = 4
N_FLAT = SEQ_LEN * BATCH

NUM_WORKERS = 32
PER_WORKER = N_FLAT // NUM_WORKERS
POS_PER_WORKER = PER_WORKER // BATCH
CHUNK = 32
NCHUNK = PER_WORKER // CHUNK
POS_PER_CHUNK = CHUNK // BATCH
NSLOT = 4
LANES = 16
VREGS_PER_ROW = D_MODEL // LANES
SCALE = math.sqrt(D_MODEL)


def _make_kernel():
    mesh = plsc.VectorSubcoreMesh(core_axis_name="c", subcore_axis_name="s")

    @functools.partial(
        pl.kernel,
        mesh=mesh,
        out_type=jax.ShapeDtypeStruct((SEQ_LEN, BATCH, D_MODEL), jnp.float32),
        scratch_types=[
            pltpu.VMEM((NCHUNK, CHUNK), jnp.int32),
            pltpu.VMEM((NSLOT, POS_PER_CHUNK, 1, D_MODEL), jnp.float32),
            pltpu.VMEM((NSLOT, CHUNK, D_MODEL), jnp.float32),
            pltpu.SemaphoreType.DMA((NSLOT,)),
            pltpu.SemaphoreType.DMA((NSLOT,)),
        ],
    )
    def k(x_hbm, table_hbm, pe_hbm, out_hbm, idx_v, pe_v, buf_v, isem, osem):
        cid = lax.axis_index("c")
        sid = lax.axis_index("s")
        wid = sid * 2 + cid
        pos0 = wid * POS_PER_WORKER

        pltpu.sync_copy(x_hbm.at[wid], idx_v)

        def fire_in(c):
            s = c % NSLOT
            g = pltpu.async_copy(table_hbm.at[idx_v.at[c]], buf_v.at[s],
                                 isem.at[s])
            p = pltpu.async_copy(
                pe_hbm.at[pl.ds(pos0 + c * POS_PER_CHUNK, POS_PER_CHUNK)],
                pe_v.at[s], isem.at[s])
            return g, p

        def compute(s):
            def j_body(j, _):
                off = j * LANES
                for p in range(POS_PER_CHUNK):
                    pvec = pe_v[s, p, 0, pl.ds(off, LANES)]
                    for b in range(BATCH):
                        r = p * BATCH + b
                        buf_v[s, r, pl.ds(off, LANES)] = (
                            buf_v[s, r, pl.ds(off, LANES)] * SCALE + pvec)
                return 0

            lax.fori_loop(0, VREGS_PER_ROW, j_body, 0)

        def fire_out(c):
            s = c % NSLOT
            handles = []
            for p in range(POS_PER_CHUNK):
                handles.append(pltpu.async_copy(
                    buf_v.at[s].at[pl.ds(p * BATCH, BATCH)],
                    out_hbm.at[pos0 + c * POS_PER_CHUNK + p],
                    osem.at[s]))
            return handles

        in_flight = {}
        store_flight = {}
        for c in range(2):
            in_flight[c] = fire_in(c)

        for c in range(NCHUNK):
            s = c % NSLOT
            if c + 2 < NCHUNK:
                if c - 2 >= 0:
                    for h in store_flight.pop(c - 2):
                        h.wait()
                in_flight[c + 2] = fire_in(c + 2)
            g, p = in_flight.pop(c)
            g.wait()
            p.wait()
            compute(s)
            store_flight[c] = fire_out(c)

        for c in (NCHUNK - 2, NCHUNK - 1):
            for h in store_flight.pop(c):
                h.wait()

    return k


_sc_kernel = _make_kernel()


def kernel(x, table, pe):
    xf = jnp.asarray(x, jnp.int32).reshape(NUM_WORKERS, NCHUNK, CHUNK)
    return _sc_kernel(xf, table, pe)


# speedup vs baseline: 2.3688x; 1.0622x over previous
import functools
import math

import jax
import jax.numpy as jnp
from jax import lax
from jax.experimental import pallas as pl
from jax.experimental.pallas import tpu as pltpu
from jax.experimental.pallas import tpu_sc as plsc

D_MODEL = 768
SEQ_LEN = 4096
BATCH = 4
N_FLAT = SEQ_LEN * BATCH

NUM_WORKERS = 32
PER_WORKER = N_FLAT // NUM_WORKERS
POS_PER_WORKER = PER_WORKER // BATCH
CHUNK = 32
NCHUNK = PER_WORKER // CHUNK
POS_PER_CHUNK = CHUNK // BATCH
NSLOT = 4
LANES = 16
VREGS_PER_ROW = D_MODEL // LANES
SCALE = math.sqrt(D_MODEL)


def _make_kernel():
    mesh = plsc.VectorSubcoreMesh(core_axis_name="c", subcore_axis_name="s")

    @functools.partial(
        pl.kernel,
        mesh=mesh,
        out_type=jax.ShapeDtypeStruct((SEQ_LEN, BATCH, D_MODEL), jnp.float32),
        scratch_types=[
            pltpu.VMEM((NCHUNK, CHUNK), jnp.int32),
            pltpu.VMEM((NSLOT, POS_PER_CHUNK, 1, D_MODEL), jnp.float32),
            pltpu.VMEM((NSLOT, CHUNK, D_MODEL), jnp.float32),
            pltpu.SemaphoreType.DMA((NSLOT,)),
            pltpu.SemaphoreType.DMA((NSLOT,)),
        ],
    )
    def k(x_hbm, table_hbm, pe_hbm, out_hbm, idx_v, pe_v, buf_v, isem, osem):
        cid = lax.axis_index("c")
        sid = lax.axis_index("s")
        wid = sid * 2 + cid
        pos0 = wid * POS_PER_WORKER
        out_flat = out_hbm.reshape(N_FLAT, D_MODEL)

        pltpu.sync_copy(x_hbm.at[wid], idx_v)

        def fire_in(c):
            s = c % NSLOT
            g = pltpu.async_copy(table_hbm.at[idx_v.at[c]], buf_v.at[s],
                                 isem.at[s])
            p = pltpu.async_copy(
                pe_hbm.at[pl.ds(pos0 + c * POS_PER_CHUNK, POS_PER_CHUNK)],
                pe_v.at[s], isem.at[s])
            return g, p

        def compute(s):
            def j_body(j, _):
                off = j * LANES
                for p in range(POS_PER_CHUNK):
                    pvec = pe_v[s, p, 0, pl.ds(off, LANES)]
                    for b in range(BATCH):
                        r = p * BATCH + b
                        buf_v[s, r, pl.ds(off, LANES)] = (
                            buf_v[s, r, pl.ds(off, LANES)] * SCALE + pvec)
                return 0

            lax.fori_loop(0, VREGS_PER_ROW, j_body, 0)

        def fire_out(c):
            s = c % NSLOT
            return [pltpu.async_copy(
                buf_v.at[s],
                out_flat.at[pl.ds(wid * PER_WORKER + c * CHUNK, CHUNK)],
                osem.at[s])]

        in_flight = {}
        store_flight = {}
        for c in range(2):
            in_flight[c] = fire_in(c)

        for c in range(NCHUNK):
            s = c % NSLOT
            if c + 2 < NCHUNK:
                if c - 2 >= 0:
                    for h in store_flight.pop(c - 2):
                        h.wait()
                in_flight[c + 2] = fire_in(c + 2)
            g, p = in_flight.pop(c)
            g.wait()
            p.wait()
            compute(s)
            store_flight[c] = fire_out(c)

        for c in (NCHUNK - 2, NCHUNK - 1):
            for h in store_flight.pop(c):
                h.wait()

    return k


_sc_kernel = _make_kernel()


def kernel(x, table, pe):
    xf = jnp.asarray(x, jnp.int32).reshape(NUM_WORKERS, NCHUNK, CHUNK)
    return _sc_kernel(xf, table, pe)
